# baseline (speedup 1.0000x reference)
import functools

import jax
import jax.numpy as jnp
from jax import lax
from jax.experimental import pallas as pl
from jax.experimental.pallas import tpu as pltpu
from jax.experimental.pallas import tpu_sc as plsc

_NUM_CLASSES = 1000
_FILTER_K = 10
_SCALE = 20.0
_B = 16384
_D = 128
_TB = 512
_CP = 1024
_NALL = _CP + _B
_NW = 16
_WW = _CP // _NW
_FW = _B // _NW


def _k1_body(feat_ref, lr_ref, la_ref, w_ref, ent_ref, pmax_ref, yhat_ref,
             agree_ref, fscale_ref):
    f = feat_ref[...]
    w = w_ref[...]
    logits = jax.lax.dot_general(f, w, (((1,), (1,)), ((), ())),
                                 preferred_element_type=jnp.float32)
    c = logits.shape[1]
    rowmax = jnp.max(logits, axis=1, keepdims=True)
    s = logits - rowmax
    es = jnp.exp(s)
    z = jnp.sum(es, axis=1, keepdims=True)
    ent = jnp.log(z) - jnp.sum(es * s, axis=1, keepdims=True) / z
    iota = jax.lax.broadcasted_iota(jnp.int32, logits.shape, 1)
    yhat = jnp.min(jnp.where(logits == rowmax, iota, c), axis=1)
    lr = lr_ref[...]
    la = la_ref[...]
    amr = jnp.min(jnp.where(lr == jnp.max(lr, axis=1, keepdims=True), iota, c),
                  axis=1)
    ama = jnp.min(jnp.where(la == jnp.max(la, axis=1, keepdims=True), iota, c),
                  axis=1)
    pmax = 1.0 / z
    fnorm = jnp.sqrt(jnp.sum(f * f, axis=1, keepdims=True))
    fscale = jnp.maximum(pmax, 1e-6) / jnp.maximum(fnorm, 1e-12)
    ent_ref[...] = ent[:, 0]
    pmax_ref[...] = pmax[:, 0]
    yhat_ref[...] = yhat
    agree_ref[...] = (amr == ama).astype(jnp.int32)
    fscale_ref[...] = fscale[:, 0]


def _k1(feat, logits_raw, logits_aug, w):
    nb = _B // _TB
    return pl.pallas_call(
        _k1_body,
        grid=(nb,),
        in_specs=[
            pl.BlockSpec((_TB, _D), lambda i: (i, 0)),
            pl.BlockSpec((_TB, _NUM_CLASSES), lambda i: (i, 0)),
            pl.BlockSpec((_TB, _NUM_CLASSES), lambda i: (i, 0)),
            pl.BlockSpec((_NUM_CLASSES, _D), lambda i: (0, 0)),
        ],
        out_specs=[pl.BlockSpec((_TB,), lambda i: (i,)) for _ in range(5)],
        out_shape=[
            jax.ShapeDtypeStruct((_B,), jnp.float32),
            jax.ShapeDtypeStruct((_B,), jnp.float32),
            jax.ShapeDtypeStruct((_B,), jnp.int32),
            jax.ShapeDtypeStruct((_B,), jnp.int32),
            jax.ShapeDtypeStruct((_B,), jnp.float32),
        ],
    )(feat, logits_raw, logits_aug, w)


def _k2_body(w_ref, ent_ref, y_ref, wscale_ref):
    w = w_ref[...]
    warm = jax.lax.dot_general(w, w, (((1,), (1,)), ((), ())),
                               preferred_element_type=jnp.float32)
    c = warm.shape[1]
    rowmax = jnp.max(warm, axis=1, keepdims=True)
    s = warm - rowmax
    es = jnp.exp(s)
    z = jnp.sum(es, axis=1, keepdims=True)
    ent = jnp.log(z) - jnp.sum(es * s, axis=1, keepdims=True) / z
    iota = jax.lax.broadcasted_iota(jnp.int32, warm.shape, 1)
    y0 = jnp.min(jnp.where(warm == rowmax, iota, c), axis=1)
    wnorm = jnp.sqrt(jnp.sum(w * w, axis=1, keepdims=True))
    wscale = jnp.maximum(1.0 / z, 1e-6) / jnp.maximum(wnorm, 1e-12)
    ent_ref[...] = ent[:, 0]
    y_ref[...] = y0
    wscale_ref[...] = wscale[:, 0]


def _k2(w):
    return pl.pallas_call(
        _k2_body,
        out_shape=[
            jax.ShapeDtypeStruct((_NUM_CLASSES,), jnp.float32),
            jax.ShapeDtypeStruct((_NUM_CLASSES,), jnp.int32),
            jax.ShapeDtypeStruct((_NUM_CLASSES,), jnp.float32),
        ],
    )(w)


def _nth_smallest_bits(bits, k):
    def step(i, prefix):
        b = 30 - i
        t = prefix | (1 << b)
        cnt = jnp.sum((bits < t).astype(jnp.int32))
        return jnp.where(cnt <= k, t, prefix)

    return jax.lax.fori_loop(0, 31, step, jnp.int32(0))


def _k3_body(ent_ref, pmax_ref, agree_ref, mask_ref, any_ref):
    ent = ent_ref[...]
    n = ent.size
    m = jnp.sum(ent) / n
    dyn_q = jnp.where(m >= 0.45, 0.25, jnp.where(m >= 0.38, 0.3, 0.4))
    conf_thr = jnp.where(m >= 0.45, 0.72, 0.62)
    idx_f = dyn_q * (n - 1.0)
    lo = jnp.floor(idx_f)
    k_lo = lo.astype(jnp.int32)
    bits = jax.lax.bitcast_convert_type(ent, jnp.int32)
    v_lo = jax.lax.bitcast_convert_type(_nth_smallest_bits(bits, k_lo),
                                        jnp.float32)
    v_hi = jax.lax.bitcast_convert_type(_nth_smallest_bits(bits, k_lo + 1),
                                        jnp.float32)
    g = idx_f - lo
    thr = v_lo * (1.0 - g) + v_hi * g
    mask = ((ent <= thr) & (agree_ref[...] != 0)
            & (pmax_ref[...] >= conf_thr))
    mask_ref[...] = mask.astype(jnp.int32)
    any_ref[...] = jnp.max(mask.astype(jnp.int32), keepdims=True).reshape(1, 1)


def _k3(ent, pmax, agree):
    mask2, anyf = pl.pallas_call(
        _k3_body,
        out_shape=[
            jax.ShapeDtypeStruct((128, 128), jnp.int32),
            jax.ShapeDtypeStruct((1, 1), jnp.int32),
        ],
    )(ent.reshape(128, 128), pmax.reshape(128, 128), agree.reshape(128, 128))
    return mask2.reshape(_B), anyf[0, 0]


def _sc_body(wpad_ref, feat_ref, cls_ref, ent_ref, scale_ref, anyv_ref,
             out_ref,
             rowbuf_v, wcls_v, wsc_v, fcls_v, fsc_v, scat_v, idx64_v,
             hist_v, ghist_v, anyv_v, acls_v, aeb_v, asc_v,
             cent_s, hist_s):
    wid = lax.axis_index("s")
    warm_base = wid * _WW
    feat_base = wid * _FW
    cand_feat_base = _CP + feat_base
    lane = lax.iota(jnp.int32, 16)
    zf = jnp.zeros((16,), jnp.float32)
    zi = jnp.zeros((16,), jnp.int32)

    def _zrow(r, _):
        for k in range(8):
            rowbuf_v[r, pl.ds(k * 16, 16)] = zf
        return 0
    lax.fori_loop(0, _WW, _zrow, 0)

    def _zh(r, _):
        hist_v[r, :] = zi
        return 0
    lax.fori_loop(0, _CP // 16, _zh, 0)
    for j in range(4):
        idx64_v[pl.ds(j * 16, 16)] = j * 16 + lane

    pltpu.sync_copy(rowbuf_v, cent_s.at[pl.ds(warm_base, _WW)])

    @pl.when(wid == 0)
    def _():
        pltpu.sync_copy(hist_v, hist_s)

    pltpu.sync_copy(cls_ref.at[pl.ds(warm_base, _WW)], wcls_v)
    pltpu.sync_copy(scale_ref.at[pl.ds(warm_base, _WW)], wsc_v)
    pltpu.sync_copy(cls_ref.at[pl.ds(cand_feat_base, _FW)], fcls_v)
    pltpu.sync_copy(scale_ref.at[pl.ds(cand_feat_base, _FW)], fsc_v)
    pltpu.sync_copy(anyv_ref, anyv_v)
    a = jnp.max(anyv_v[...])

    plsc.subcore_barrier()

    @pl.when(a != 0)
    def _with_mask():
        def _hw(j, _):
            sl = pl.ds(j * 16, 16)
            cls = wcls_v[sl]
            plsc.addupdate_scatter(hist_v, [cls >> 4, cls & 15],
                                   jnp.ones((16,), jnp.int32),
                                   mask=wsc_v[sl] > 0.0)
            return 0
        lax.fori_loop(0, _WW // 16, _hw, 0)

        def _hf(j, _):
            sl = pl.ds(j * 16, 16)
            cls = fcls_v[sl]
            plsc.addupdate_scatter(hist_v, [cls >> 4, cls & 15],
                                   jnp.ones((16,), jnp.int32),
                                   mask=fsc_v[sl] > 0.0)
            return 0
        lax.fori_loop(0, _FW // 16, _hf, 0)

        pltpu.sync_copy(hist_v, hist_s.at[idx64_v], add=True)
        plsc.subcore_barrier()
        pltpu.sync_copy(hist_s, ghist_v)

        def _mx(j, mx):
            return jnp.maximum(mx, jnp.max(ghist_v[j, :]))
        maxc = lax.fori_loop(0, _CP // 16, _mx, jnp.int32(0))

        @pl.when(maxc > _FILTER_K)
        def _tier3():
            pltpu.sync_copy(cls_ref, acls_v)
            pltpu.sync_copy(ent_ref, aeb_v)
            pltpu.sync_copy(scale_ref, asc_v)

            def _rank_keep(gk):
                sp = jnp.full((16,), gk, jnp.int32)
                ck = plsc.load_gather(acls_v, [sp])
                ekb = plsc.bitcast(plsc.load_gather(aeb_v, [sp]), jnp.int32)

                def _in(j, acc):
                    sl = pl.ds(j * 16, 16)
                    cj = acls_v[sl]
                    ej = plsc.bitcast(aeb_v[sl], jnp.int32)
                    vj = asc_v[sl] > 0.0
                    gj = j * 16 + lane
                    ltk = (ej < ekb) | ((ej == ekb) & (gj < gk))
                    mm = vj & (cj == ck) & ltk
                    return acc + mm.astype(jnp.int32)

                acc = lax.fori_loop(0, _NALL // 16, _in, zi)
                return (jnp.sum(acc) < _FILTER_K).astype(jnp.float32)

            m0 = lane == 0

            def _rw(i, _):
                keep = _rank_keep(warm_base + i)
                sp = jnp.full((16,), i, jnp.int32)
                old = plsc.load_gather(wsc_v, [sp])
                plsc.store_scatter(wsc_v, [sp], old * keep, mask=m0)
                return 0
            lax.fori_loop(0, _WW, _rw, 0)

            def _rf(i, _):
                keep = _rank_keep(cand_feat_base + i)
                sp = jnp.full((16,), i, jnp.int32)
                old = plsc.load_gather(fsc_v, [sp])
                plsc.store_scatter(fsc_v, [sp], old * keep, mask=m0)
                return 0
            lax.fori_loop(0, _FW, _rf, 0)

        def _fchunk(t, _):
            pltpu.sync_copy(feat_ref.at[pl.ds(feat_base + t * _WW, _WW)],
                            rowbuf_v)

            def _sr(r, _2):
                sc = plsc.load_gather(
                    fsc_v, [jnp.full((16,), t * _WW + r, jnp.int32)])
                for k in range(8):
                    sl = pl.ds(k * 16, 16)
                    rowbuf_v[r, sl] = rowbuf_v[r, sl] * sc
                return 0
            lax.fori_loop(0, _WW, _sr, 0)

            for k in range(_WW // 16):
                scat_v[pl.ds(k * 16, 16)] = fcls_v[pl.ds(t * _WW + k * 16, 16)]
            pltpu.sync_copy(rowbuf_v, cent_s.at[scat_v], add=True)
            return 0
        lax.fori_loop(0, _FW // _WW, _fchunk, 0)

    pltpu.sync_copy(wpad_ref.at[pl.ds(warm_base, _WW)], rowbuf_v)

    def _swr(r, _):
        sc = plsc.load_gather(wsc_v, [jnp.full((16,), r, jnp.int32)])
        for k in range(8):
            sl = pl.ds(k * 16, 16)
            rowbuf_v[r, sl] = rowbuf_v[r, sl] * sc
        return 0
    lax.fori_loop(0, _WW, _swr, 0)
    pltpu.sync_copy(rowbuf_v, cent_s.at[wcls_v], add=True)

    plsc.subcore_barrier()
    pltpu.sync_copy(cent_s.at[pl.ds(warm_base, _WW)],
                    out_ref.at[pl.ds(warm_base, _WW)])


def _sc_centroids(wpad, feat, clsall, entall, scaleall, anyv):
    mesh = plsc.VectorSubcoreMesh(core_axis_name="c", subcore_axis_name="s",
                                  num_cores=1)
    kern = pl.kernel(
        _sc_body,
        out_type=jax.ShapeDtypeStruct((_CP, _D), jnp.float32),
        mesh=mesh,
        scratch_types=[
            pltpu.VMEM((_WW, _D), jnp.float32),
            pltpu.VMEM((_WW,), jnp.int32),
            pltpu.VMEM((_WW,), jnp.float32),
            pltpu.VMEM((_FW,), jnp.int32),
            pltpu.VMEM((_FW,), jnp.float32),
            pltpu.VMEM((_WW,), jnp.int32),
            pltpu.VMEM((_WW,), jnp.int32),
            pltpu.VMEM((_CP // 16, 16), jnp.int32),
            pltpu.VMEM((_CP // 16, 16), jnp.int32),
            pltpu.VMEM((16,), jnp.int32),
            pltpu.VMEM((_NALL,), jnp.int32),
            pltpu.VMEM((_NALL,), jnp.float32),
            pltpu.VMEM((_NALL,), jnp.float32),
            pltpu.VMEM_SHARED((_CP, _D), jnp.float32),
            pltpu.VMEM_SHARED((_CP // 16, 16), jnp.int32),
        ],
        compiler_params=pltpu.CompilerParams(needs_layout_passes=False),
    )
    return kern(wpad, feat, clsall, entall, scaleall, anyv)


def _k5_body(feat_ref, cent_ref, out_ref):
    f = feat_ref[...]
    fn = f / jnp.maximum(
        jnp.sqrt(jnp.sum(f * f, axis=1, keepdims=True)), 1e-12)
    cent = cent_ref[...][:_NUM_CLASSES]
    cn = cent / jnp.maximum(
        jnp.sqrt(jnp.sum(cent * cent, axis=1, keepdims=True)), 1e-12)
    sim = jax.lax.dot_general(fn, cn, (((1,), (1,)), ((), ())),
                              preferred_element_type=jnp.float32)
    out_ref[...] = _SCALE * sim


def _k5(feat, cents):
    nb = _B // _TB
    return pl.pallas_call(
        _k5_body,
        grid=(nb,),
        in_specs=[
            pl.BlockSpec((_TB, _D), lambda i: (i, 0)),
            pl.BlockSpec((_CP, _D), lambda i: (0, 0)),
        ],
        out_specs=pl.BlockSpec((_TB, _NUM_CLASSES), lambda i: (i, 0)),
        out_shape=jax.ShapeDtypeStruct((_B, _NUM_CLASSES), jnp.float32),
    )(feat, cents)


def kernel(feat, logits_raw, logits_aug, W, b):
    del b
    ent, pmax, yhat, agree, fscale = _k1(feat, logits_raw, logits_aug, W)
    ents0, y0, wscale = _k2(W)
    mask, any_mask = _k3(ent, pmax, agree)

    npad = _CP - _NUM_CLASSES
    zi = jnp.zeros((npad,), jnp.int32)
    zflt = jnp.zeros((npad,), jnp.float32)
    clsall = jnp.concatenate([y0, zi, yhat])
    entall = jnp.concatenate([ents0, zflt, ent])
    scaleall = jnp.concatenate([wscale, zflt,
                                fscale * mask.astype(jnp.float32)])
    wpad = jnp.concatenate([W, jnp.zeros((npad, _D), jnp.float32)])
    anyv = jnp.full((16,), any_mask, jnp.int32)
    cents = _sc_centroids(wpad, feat, clsall, entall, scaleall, anyv)
    return _k5(feat, cents)
